# baseline (speedup 1.0000x reference)
import functools

import jax
import jax.numpy as jnp
from jax import lax
from jax.experimental import pallas as pl
from jax.experimental.pallas import tpu as pltpu
from jax.experimental.pallas import tpu_sc as plsc

_B = 4096
_T = 200
_D = 32
_BLK = 128


def _make_kernel(NC: int, NS: int):
    NW = NC * NS
    assert _B // NW == _BLK

    mesh = plsc.VectorSubcoreMesh(core_axis_name="c", subcore_axis_name="s")

    @functools.partial(
        pl.kernel,
        mesh=mesh,
        compiler_params=pltpu.CompilerParams(use_tc_tiling_on_sc=False,
                                             needs_layout_passes=False),
        out_type=jax.ShapeDtypeStruct((_T, _D // 8, _B // _BLK, 8, _BLK),
                                      jnp.float32),
        scratch_types=[
            pltpu.VMEM((_BLK, _T), jnp.int32),
            pltpu.VMEM((2, _BLK), jnp.int32),
            pltpu.VMEM((2, _BLK, _D), jnp.float32),
            pltpu.VMEM((2, _D // 8, 8, _BLK), jnp.float32),
            pltpu.SemaphoreType.DMA((2,)),
            pltpu.SemaphoreType.DMA((2,)),
        ],
    )
    def k(x_hbm, table_hbm, out_hbm, idxv, sl, gbuf, obuf, s_g, s_o):
        wid = lax.axis_index("s") * NC + lax.axis_index("c")
        pltpu.sync_copy(x_hbm.at[pl.ds(wid * _BLK, _BLK), :], idxv)

        lane = lax.iota(jnp.int32, 16)
        bi_vecs = [lane + 16 * j for j in range(8)]

        def build_sl(t, b):
            tv = jnp.zeros((16,), jnp.int32) + t
            for j in range(8):
                v = plsc.load_gather(idxv, [bi_vecs[j], tv])
                sl[b, pl.ds(16 * j, 16)] = v

        def start_gather(b):
            return pltpu.async_copy(
                table_hbm.at[sl.at[b]], gbuf.at[b], s_g.at[b])

        def wait_gather(b):
            pltpu.make_async_copy(
                table_hbm.at[sl.at[b]], gbuf.at[b], s_g.at[b]).wait()

        def start_store(t, b):
            return pltpu.async_copy(
                obuf.at[b], out_hbm.at[t, :, wid, :, :], s_o.at[b])

        def wait_store(t, b):
            pltpu.make_async_copy(
                obuf.at[b], out_hbm.at[t, :, wid, :, :], s_o.at[b]).wait()

        def repack(b):
            gb = gbuf.at[b]
            for g in range(_D // 8):
                for di in range(8):
                    dv = jnp.zeros((16,), jnp.int32) + (8 * g + di)
                    for j in range(8):
                        v = plsc.load_gather(gb, [bi_vecs[j], dv])
                        obuf[b, g, di, pl.ds(16 * j, 16)] = v

        build_sl(0, 0)
        start_gather(0)

        def body(i, carry):
            b = i % 2
            b1 = (i + 1) % 2

            @pl.when(i + 1 < _T)
            def _():
                build_sl(i + 1, b1)
                start_gather(b1)

            wait_gather(b)

            @pl.when(i >= 2)
            def _():
                wait_store(i - 2, b)

            repack(b)
            start_store(i, b)
            return carry

        lax.fori_loop(0, _T, body, 0)
        wait_store(_T - 2, 0)
        wait_store(_T - 1, 1)

    return k


def kernel(x, table):
    info = plsc.get_sparse_core_info()
    k = _make_kernel(info.num_cores, info.num_subcores)
    out5 = k(x, table)
    return out5.transpose((2, 4, 0, 1, 3)).reshape(_B, _T, _D)


# speedup vs baseline: 1.1635x; 1.1635x over previous
import functools

import jax
import jax.numpy as jnp
from jax import lax
from jax.experimental import pallas as pl
from jax.experimental.pallas import tpu as pltpu
from jax.experimental.pallas import tpu_sc as plsc

_B = 4096
_T = 200
_D = 32
_BLK = 128


def _make_kernel(NC: int, NS: int):
    NW = NC * NS
    assert _B // NW == _BLK

    mesh = plsc.VectorSubcoreMesh(core_axis_name="c", subcore_axis_name="s")

    @functools.partial(
        pl.kernel,
        mesh=mesh,
        compiler_params=pltpu.CompilerParams(use_tc_tiling_on_sc=False,
                                             needs_layout_passes=False),
        out_type=jax.ShapeDtypeStruct((_T, _D // 8, _B // _BLK, 8 * _BLK),
                                      jnp.float32),
        scratch_types=[
            pltpu.VMEM((_BLK, _T), jnp.int32),
            pltpu.VMEM((2, _BLK), jnp.int32),
            pltpu.VMEM((2, _BLK, _D), jnp.float32),
            pltpu.VMEM((2, _D // 8, 8 * _BLK), jnp.float32),
            pltpu.SemaphoreType.DMA((2,)),
            pltpu.SemaphoreType.DMA((2,)),
        ],
    )
    def k(x_hbm, table_hbm, out_hbm, idxv, sl, gbuf, obuf, s_g, s_o):
        wid = lax.axis_index("s") * NC + lax.axis_index("c")
        pltpu.sync_copy(x_hbm.at[pl.ds(wid * _BLK, _BLK), :], idxv)

        lane = lax.iota(jnp.int32, 16)

        def build_sl(t, b):
            tv = jnp.zeros((16,), jnp.int32) + t

            @plsc.parallel_loop(0, 8, unroll=8)
            def _(j):
                bi = lane + j * 16
                v = plsc.load_gather(idxv, [bi, tv])
                sl[b, pl.ds(j * 16, 16)] = v

        def start_gather(b):
            return pltpu.async_copy(
                table_hbm.at[sl.at[b]], gbuf.at[b], s_g.at[b])

        def wait_gather(b):
            pltpu.make_async_copy(
                table_hbm.at[sl.at[b]], gbuf.at[b], s_g.at[b]).wait()

        def start_store(t, b):
            return pltpu.async_copy(
                obuf.at[b], out_hbm.at[t, :, wid, :], s_o.at[b])

        def wait_store(t, b):
            pltpu.make_async_copy(
                obuf.at[b], out_hbm.at[t, :, wid, :], s_o.at[b]).wait()

        def repack(b):
            gb = gbuf.at[b]
            for g in range(_D // 8):
                @plsc.parallel_loop(0, 64, unroll=8)
                def _(k2):
                    di = k2 >> 3
                    j = k2 & 7
                    bi = lane + j * 16
                    dv = jnp.zeros((16,), jnp.int32) + (8 * g + di)
                    v = plsc.load_gather(gb, [bi, dv])
                    obuf[b, g, pl.ds(k2 * 16, 16)] = v

        build_sl(0, 0)
        start_gather(0)

        def body(i, carry):
            b = i % 2
            b1 = (i + 1) % 2

            @pl.when(i + 1 < _T)
            def _():
                build_sl(i + 1, b1)
                start_gather(b1)

            wait_gather(b)

            @pl.when(i >= 2)
            def _():
                wait_store(i - 2, b)

            repack(b)
            start_store(i, b)
            return carry

        lax.fori_loop(0, _T, body, 0)
        wait_store(_T - 2, 0)
        wait_store(_T - 1, 1)

    return k


def kernel(x, table):
    info = plsc.get_sparse_core_info()
    k = _make_kernel(info.num_cores, info.num_subcores)
    out5 = k(x, table)
    out5 = out5.reshape(_T, _D // 8, _B // _BLK, 8, _BLK)
    return out5.transpose((2, 4, 0, 1, 3)).reshape(_B, _T, _D)
